# baseline (speedup 1.0000x reference)
import functools

import jax
import jax.numpy as jnp
from jax import lax
from jax.experimental import pallas as pl
from jax.experimental.pallas import tpu as pltpu
from jax.experimental.pallas import tpu_sc as plsc

_L = 16
_NBUF = 4


def _sc_time_encode(x, t2, tpad, H):
    (NH,) = x.shape
    N = NH // H
    NW = 32
    per_w = N // NW
    C = 160
    n_chunks = per_w // C
    assert per_w % C == 0 and n_chunks % _NBUF == 0 and C % _L == 0
    mesh = plsc.VectorSubcoreMesh(core_axis_name="c", subcore_axis_name="s")

    @functools.partial(
        pl.kernel,
        out_type=jax.ShapeDtypeStruct((NH,), jnp.float32),
        mesh=mesh,
        scratch_types=[
            pltpu.VMEM((tpad.shape[0],), jnp.float32),
            [pltpu.VMEM((C,), jnp.int32) for _ in range(_NBUF)],
            [pltpu.VMEM((C * H,), jnp.float32) for _ in range(_NBUF)],
            pltpu.SemaphoreType.DMA((_NBUF,)),
            pltpu.SemaphoreType.DMA((_NBUF,)),
            pltpu.SemaphoreType.DMA((_NBUF,)),
        ],
    )
    def k(x_hbm, t_hbm, tab_hbm, out_hbm, tab_v, idx_v, buf_v, sx, st, so):
        wid = lax.axis_index("s") * 2 + lax.axis_index("c")
        base = wid * per_w
        pltpu.sync_copy(tab_hbm, tab_v)

        def in_copies(g, b):
            row0 = base + g * C
            return (
                pltpu.make_async_copy(x_hbm.at[pl.ds(row0 * H, C * H)],
                                      buf_v[b], sx.at[b]),
                pltpu.make_async_copy(t_hbm.at[pl.ds(row0, C)], idx_v[b],
                                      st.at[b]),
            )

        def out_copy(g, b):
            row0 = base + g * C
            return pltpu.make_async_copy(buf_v[b],
                                         out_hbm.at[pl.ds(row0 * H, C * H)],
                                         so.at[b])

        for g in range(_NBUF - 1):
            for cp in in_copies(g, g):
                cp.start()

        def outer(i, _):
            for b in range(_NBUF):
                g = i * _NBUF + b
                for cp in in_copies(g, b):
                    cp.wait()

                def grp_body(gr, _, b=b):
                    r0 = gr * _L
                    tvec = idx_v[b][pl.ds(r0, _L)]
                    for ii in range(_L):
                        t = tvec[ii]
                        for j in range(H // _L):
                            plsc.addupdate(
                                buf_v[b].at[pl.ds((r0 + ii) * H + j * _L, _L)],
                                tab_v[pl.ds(t * H + j * _L, _L)],
                            )
                    return ()

                lax.fori_loop(0, C // _L, grp_body, ())
                out_copy(g, b).start()

                gp = g + _NBUF - 1
                b2 = (b + _NBUF - 1) % _NBUF

                @pl.when(gp < n_chunks)
                def _(g=g, gp=gp, b2=b2):
                    @pl.when(g >= 1)
                    def _():
                        out_copy(g - 1, b2).wait()

                    for cp in in_copies(gp, b2):
                        cp.start()

            return ()

        lax.fori_loop(0, n_chunks // _NBUF, outer, ())

        for g in range(n_chunks - _NBUF, n_chunks):
            out_copy(g, g % _NBUF).wait()

    return k(x, t2, tpad)


def kernel(inputs, times, table):
    B, L, H = inputs.shape
    NP = table.shape[0]
    N = B * L

    TROWS = 32
    x = inputs.reshape(N * H)
    tpad = jnp.zeros((TROWS, H), jnp.float32).at[:NP].set(table).reshape(-1)
    t2 = times.astype(jnp.int32).at[:, 0].set(TROWS - 1).reshape(N)

    out = _sc_time_encode(x, t2, tpad, H)
    return out.reshape(B, L, H)


# speedup vs baseline: 2.0837x; 2.0837x over previous
import functools

import jax
import jax.numpy as jnp
from jax import lax
from jax.experimental import pallas as pl
from jax.experimental.pallas import tpu as pltpu
from jax.experimental.pallas import tpu_sc as plsc

_L = 16
_NBUF = 3


def _sc_time_encode(x, t2, tpad):
    N, H = x.shape
    TROWS = tpad.shape[0]
    NW = 32
    per_w = N // NW
    C = min(256, per_w)
    n_chunks = per_w // C
    nbuf = min(_NBUF, n_chunks)
    mesh = plsc.VectorSubcoreMesh(core_axis_name="c", subcore_axis_name="s")

    @functools.partial(
        pl.kernel,
        out_type=jax.ShapeDtypeStruct((N, H), jnp.float32),
        mesh=mesh,
        scratch_types=[
            pltpu.VMEM((TROWS, H), jnp.float32),
            pltpu.VMEM((nbuf, C), jnp.int32),
            pltpu.VMEM((nbuf, C, H), jnp.float32),
            pltpu.SemaphoreType.DMA((nbuf,)),
            pltpu.SemaphoreType.DMA((nbuf,)),
            pltpu.SemaphoreType.DMA((nbuf,)),
        ],
    )
    def k(x_hbm, t_hbm, tab_hbm, out_hbm, tab_v, idx_v, buf_v, sx, st, so):
        wid = lax.axis_index("s") * 2 + lax.axis_index("c")
        base = wid * per_w
        pltpu.sync_copy(tab_hbm, tab_v)

        def in_copies(g, b):
            row0 = base + g * C
            return (
                pltpu.make_async_copy(x_hbm.at[pl.ds(row0, C)], buf_v.at[b],
                                      sx.at[b]),
                pltpu.make_async_copy(t_hbm.at[pl.ds(row0, C)], idx_v.at[b],
                                      st.at[b]),
            )

        def out_copy(g, b):
            row0 = base + g * C
            return pltpu.make_async_copy(buf_v.at[b], out_hbm.at[pl.ds(row0, C)],
                                         so.at[b])

        for g in range(nbuf - 1):
            for cp in in_copies(g, g % nbuf):
                cp.start()

        def step(g, _):
            b = lax.rem(g, nbuf)
            for cp in in_copies(g, b):
                cp.wait()

            @plsc.parallel_loop(0, C // _L, 1, unroll=2)
            def grp_body(gr):
                r0 = gr * _L
                tvec = idx_v[b, pl.ds(r0, _L)]
                for i in range(_L):
                    t = tvec[i]
                    rows = [tab_v[t, pl.ds(j * _L, _L)] for j in range(H // _L)]
                    for j in range(H // _L):
                        plsc.addupdate(
                            buf_v.at[b, r0 + i, pl.ds(j * _L, _L)], rows[j])

            out_copy(g, b).start()

            gp = g + nbuf - 1

            @pl.when(gp < n_chunks)
            def _():
                b2 = lax.rem(gp, nbuf)

                @pl.when(g >= 1)
                def _():
                    out_copy(g - 1, b2).wait()

                for cp in in_copies(gp, b2):
                    cp.start()

            return ()

        lax.fori_loop(0, n_chunks, step, ())

        for g in range(max(0, n_chunks - nbuf), n_chunks):
            out_copy(g, g % nbuf).wait()

    return k(x, t2, tpad)


def kernel(inputs, times, table):
    B, L, H = inputs.shape
    NP = table.shape[0]
    N = B * L

    TROWS = 32
    x = inputs.reshape(N, H)
    tpad = jnp.zeros((TROWS, H), jnp.float32).at[:NP].set(table)
    t2 = times.astype(jnp.int32).at[:, 0].set(TROWS - 1).reshape(N)

    out = _sc_time_encode(x, t2, tpad)
    return out.reshape(B, L, H)
